# baseline (speedup 1.0000x reference)
import jax
import jax.numpy as jnp
from jax import lax
from jax.experimental import pallas as pl
from jax.experimental.pallas import tpu as pltpu
from jax.experimental.pallas import tpu_sc as plsc

VOCAB = 100000
EMBED = 64
BATCH = 4096
SEQ = 200
TOTAL = BATCH * SEQ

_info = plsc.get_sparse_core_info()
NC, NS = _info.num_cores, _info.num_subcores
NW = NC * NS
B_PER_W = TOTAL // NW
CHUNK = 256
NCHUNK = B_PER_W // CHUNK
NBUF = 4
NROUND = NCHUNK // NBUF


def _body(ids_hbm, table_hbm, out_hbm, idx_v, rows, sems_g, sems_w):
    wid = lax.axis_index("s") * NC + lax.axis_index("c")
    base = wid * B_PER_W
    pltpu.sync_copy(ids_hbm.at[pl.ds(base, B_PER_W)], idx_v)

    def gather(j, b):
        idx_slice = idx_v.at[pl.ds(j * CHUNK, CHUNK)]
        pltpu.async_copy(table_hbm.at[idx_slice], rows[b], sems_g[b])

    def gather_wait(b):
        idx_slice = idx_v.at[pl.ds(0, CHUNK)]
        pltpu.make_async_copy(table_hbm.at[idx_slice], rows[b], sems_g[b]).wait()

    def write(j, b):
        pltpu.async_copy(
            rows[b], out_hbm.at[pl.ds(base + j * CHUNK, CHUNK)], sems_w[b]
        )

    def write_wait(b):
        pltpu.make_async_copy(
            rows[b], out_hbm.at[pl.ds(base, CHUNK)], sems_w[b]
        ).wait()

    for b in range(NBUF):
        gather(b, b)

    def round_(g, carry):
        j0 = g * NBUF
        for b in range(NBUF):
            gather_wait(b)
            write(j0 + b, b)
        @pl.when(g + 1 < NROUND)
        def _refill():
            for b in range(NBUF):
                write_wait(b)
                gather(j0 + NBUF + b, b)
        return carry

    lax.fori_loop(0, NROUND, round_, 0)
    for b in range(NBUF):
        write_wait(b)


@jax.jit
def kernel(images_ids, embedding_weight):
    ids_flat = images_ids.reshape(TOTAL)
    mesh = plsc.VectorSubcoreMesh(core_axis_name="c", subcore_axis_name="s")

    def body(ids_hbm, table_hbm, out_hbm, idx_v, *bufs_and_sems):
        rows = bufs_and_sems[:NBUF]
        sems_g = bufs_and_sems[NBUF : 2 * NBUF]
        sems_w = bufs_and_sems[2 * NBUF :]
        _body(ids_hbm, table_hbm, out_hbm, idx_v, rows, sems_g, sems_w)

    out = pl.kernel(
        body,
        out_type=jax.ShapeDtypeStruct((TOTAL, EMBED), jnp.float32),
        mesh=mesh,
        scratch_types=(
            [pltpu.VMEM((B_PER_W,), jnp.int32)]
            + [pltpu.VMEM((CHUNK, EMBED), jnp.float32) for _ in range(NBUF)]
            + [pltpu.SemaphoreType.DMA for _ in range(2 * NBUF)]
        ),
        compiler_params=pltpu.CompilerParams(use_tc_tiling_on_sc=False),
    )(ids_flat, embedding_weight)
    return out.reshape(BATCH, SEQ, EMBED)


# speedup vs baseline: 1.0045x; 1.0045x over previous
import jax
import jax.numpy as jnp
from jax import lax
from jax.experimental import pallas as pl
from jax.experimental.pallas import tpu as pltpu
from jax.experimental.pallas import tpu_sc as plsc

VOCAB = 100000
EMBED = 64
BATCH = 4096
SEQ = 200

_info = plsc.get_sparse_core_info()
NC, NS = _info.num_cores, _info.num_subcores
NW = NC * NS
ROWS_PER_W = BATCH // NW
NBUF = 4
NROUND = ROWS_PER_W // NBUF


def _body(ids_hbm, table_hbm, out_hbm, idx_v, rows, sems_g, sems_w):
    wid = lax.axis_index("s") * NC + lax.axis_index("c")
    base = wid * ROWS_PER_W
    pltpu.sync_copy(ids_hbm.at[pl.ds(base, ROWS_PER_W)], idx_v)

    def gather(r, b):
        pltpu.async_copy(table_hbm.at[idx_v.at[r]], rows[b], sems_g[b])

    def gather_wait(b):
        pltpu.make_async_copy(table_hbm.at[idx_v.at[0]], rows[b], sems_g[b]).wait()

    def write(r, b):
        pltpu.async_copy(rows[b], out_hbm.at[base + r], sems_w[b])

    def write_wait(b):
        pltpu.make_async_copy(rows[b], out_hbm.at[base], sems_w[b]).wait()

    for b in range(NBUF):
        gather(b, b)

    def round_(g, carry):
        r0 = g * NBUF
        for b in range(NBUF):
            gather_wait(b)
            write(r0 + b, b)
        @pl.when(g + 1 < NROUND)
        def _refill():
            for b in range(NBUF):
                write_wait(b)
                gather(r0 + NBUF + b, b)
        return carry

    lax.fori_loop(0, NROUND, round_, 0)
    for b in range(NBUF):
        write_wait(b)


@jax.jit
def kernel(images_ids, embedding_weight):
    mesh = plsc.VectorSubcoreMesh(core_axis_name="c", subcore_axis_name="s")

    def body(ids_hbm, table_hbm, out_hbm, idx_v, *bufs_and_sems):
        rows = bufs_and_sems[:NBUF]
        sems_g = bufs_and_sems[NBUF : 2 * NBUF]
        sems_w = bufs_and_sems[2 * NBUF :]
        _body(ids_hbm, table_hbm, out_hbm, idx_v, rows, sems_g, sems_w)

    return pl.kernel(
        body,
        out_type=jax.ShapeDtypeStruct((BATCH, SEQ, EMBED), jnp.float32),
        mesh=mesh,
        scratch_types=(
            [pltpu.VMEM((ROWS_PER_W, SEQ), jnp.int32)]
            + [pltpu.VMEM((SEQ, EMBED), jnp.float32) for _ in range(NBUF)]
            + [pltpu.SemaphoreType.DMA for _ in range(2 * NBUF)]
        ),
        compiler_params=pltpu.CompilerParams(use_tc_tiling_on_sc=False),
    )(images_ids, embedding_weight)
